# baseline (speedup 1.0000x reference)
import functools

import jax
import jax.numpy as jnp
from jax import lax
from jax.experimental import pallas as pl
from jax.experimental.pallas import tpu as pltpu
from jax.experimental.pallas import tpu_sc as plsc

_VOCAB = 1000000
_EMB = 32
_B = 16384
_L = 50
_N = _B * _L

_NC = 2
_NS = 16
_NW = _NC * _NS
_PW = _N // _NW
_C = 512
_NCHUNK = _PW // _C
_BPL = _B // _C

_LANES = 16

_mesh = plsc.VectorSubcoreMesh(core_axis_name="c", subcore_axis_name="s")


@functools.partial(
    pl.kernel,
    out_type=jax.ShapeDtypeStruct((_L, _EMB, _B), jnp.float32),
    mesh=_mesh,
    scratch_types=[
        pltpu.VMEM((_PW,), jnp.int32),
        pltpu.VMEM((_C, _EMB), jnp.float32),
        pltpu.VMEM((_C, _EMB), jnp.float32),
        pltpu.VMEM((_EMB, _C), jnp.float32),
        pltpu.VMEM((_EMB, _C), jnp.float32),
        pltpu.SemaphoreType.DMA,
        pltpu.SemaphoreType.DMA,
        pltpu.SemaphoreType.DMA,
        pltpu.SemaphoreType.DMA,
    ],
    compiler_params=pltpu.CompilerParams(
        use_tc_tiling_on_sc=False, needs_layout_passes=False
    ),
)
def _gather(idx_hbm, table_hbm, out_t, idx_v, rows0, rows1, trows0, trows1,
            gsem0, gsem1, osem0, osem1):
    wid = lax.axis_index("s") * _NC + lax.axis_index("c")
    base = wid * _PW
    t0 = wid * _NCHUNK
    rows = (rows0, rows1)
    trows = (trows0, trows1)
    gsems = (gsem0, gsem1)
    osems = (osem0, osem1)

    pltpu.sync_copy(idx_hbm.at[pl.ds(base, _PW)], idx_v)

    def _gat(c, b):
        return pltpu.make_async_copy(
            table_hbm.at[idx_v.at[pl.ds(c * _C, _C)]], rows[b], gsems[b]
        )

    def _out(c, b):
        t = t0 + c
        l = t // _BPL
        b0 = (t % _BPL) * _C
        return pltpu.make_async_copy(
            trows[b], out_t.at[l, :, pl.ds(b0, _C)], osems[b]
        )

    def _transpose(b):
        lane = lax.iota(jnp.int32, _LANES)
        cids = [lane * 0 + e for e in range(_EMB)]

        @pl.loop(0, _C // _LANES, unroll=4)
        def _blk(j):
            ridx = j * _LANES + lane
            for e in range(_EMB):
                v = plsc.load_gather(rows[b], [ridx, cids[e]])
                trows[b][e, pl.ds(j * _LANES, _LANES)] = v

    _gat(0, 0).start()

    @pl.loop(0, _NCHUNK, step=2)
    def _grp(g):
        for b in range(2):
            c = g + b

            @pl.when(c >= 2)
            def _():
                _out(c - 2, b).wait()

            @pl.when(c + 1 < _NCHUNK)
            def _():
                _gat(c + 1, 1 - b).start()

            _gat(c, b).wait()
            _transpose(b)
            plsc.subcore_barrier()
            _out(c, b).start()

    _out(_NCHUNK - 2, 0).wait()
    _out(_NCHUNK - 1, 1).wait()


def kernel(x, input_emb):
    res = _gather(x.T.reshape(_N), input_emb)
    return jnp.transpose(res, (2, 0, 1))


# speedup vs baseline: 1.5094x; 1.5094x over previous
import functools

import jax
import jax.numpy as jnp
from jax import lax
from jax.experimental import pallas as pl
from jax.experimental.pallas import tpu as pltpu
from jax.experimental.pallas import tpu_sc as plsc

_VOCAB = 1000000
_EMB = 32
_B = 16384
_L = 50
_N = _B * _L

_NC = 2
_NS = 16
_NW = _NC * _NS
_PW = _N // _NW
_C = 512
_NCHUNK = _PW // _C
_BPL = _B // _C

_LANES = 16

_mesh = plsc.VectorSubcoreMesh(core_axis_name="c", subcore_axis_name="s")


@functools.partial(
    pl.kernel,
    out_type=jax.ShapeDtypeStruct((_L, _EMB, _B), jnp.float32),
    mesh=_mesh,
    scratch_types=[
        pltpu.VMEM((_PW,), jnp.int32),
        pltpu.VMEM((_C, _EMB), jnp.float32),
        pltpu.VMEM((_C, _EMB), jnp.float32),
        pltpu.VMEM((_EMB, _C), jnp.float32),
        pltpu.VMEM((_EMB, _C), jnp.float32),
        pltpu.SemaphoreType.DMA,
        pltpu.SemaphoreType.DMA,
        pltpu.SemaphoreType.DMA,
        pltpu.SemaphoreType.DMA,
    ],
    compiler_params=pltpu.CompilerParams(
        use_tc_tiling_on_sc=False, needs_layout_passes=False
    ),
)
def _gather(idx_hbm, table_hbm, out_t, idx_v, rows0, rows1, trows0, trows1,
            gsem0, gsem1, osem0, osem1):
    wid = lax.axis_index("s") * _NC + lax.axis_index("c")
    base = wid * _PW
    t0 = wid * _NCHUNK
    rows = (rows0, rows1)
    trows = (trows0, trows1)
    gsems = (gsem0, gsem1)
    osems = (osem0, osem1)

    pltpu.sync_copy(idx_hbm.at[pl.ds(base, _PW)], idx_v)

    def _gat(c, b):
        return pltpu.make_async_copy(
            table_hbm.at[idx_v.at[pl.ds(c * _C, _C)]], rows[b], gsems[b]
        )

    def _out(c, b):
        t = t0 + c
        l = t // _BPL
        b0 = (t % _BPL) * _C
        return pltpu.make_async_copy(
            trows[b], out_t.at[l, :, pl.ds(b0, _C)], osems[b]
        )

    def _transpose(b):
        lane = lax.iota(jnp.int32, _LANES)
        cids = [(lane + d) & (_EMB - 1) for d in range(_EMB)]

        @pl.loop(0, _C // _LANES, unroll=2)
        def _blk(j):
            ridx = j * _LANES + lane
            for d in range(_EMB):
                v = plsc.load_gather(rows[b], [ridx, cids[d]])
                plsc.store_scatter(trows[b], [cids[d], ridx], v)

    _gat(0, 0).start()

    @pl.loop(0, _NCHUNK, step=2)
    def _grp(g):
        for b in range(2):
            c = g + b

            @pl.when(c >= 2)
            def _():
                _out(c - 2, b).wait()

            @pl.when(c + 1 < _NCHUNK)
            def _():
                _gat(c + 1, 1 - b).start()

            _gat(c, b).wait()
            _transpose(b)
            plsc.subcore_barrier()
            _out(c, b).start()

    _out(_NCHUNK - 2, 0).wait()
    _out(_NCHUNK - 1, 1).wait()


def kernel(x, input_emb):
    res = _gather(x.T.reshape(_N), input_emb)
    return jnp.transpose(res, (2, 0, 1))
